# baseline (speedup 1.0000x reference)
import functools

import jax
import jax.numpy as jnp
from jax import lax
from jax.experimental import pallas as pl
from jax.experimental.pallas import tpu as pltpu
from jax.experimental.pallas import tpu_sc as plsc

_B = 16384
_D = 64


@functools.cache
def _build_gather(num_cores: int, num_subcores: int):
    nw = num_cores * num_subcores
    b_per_w = _B // nw
    mesh = plsc.VectorSubcoreMesh(core_axis_name="c", subcore_axis_name="s")

    @functools.partial(
        pl.kernel,
        mesh=mesh,
        out_type=jax.ShapeDtypeStruct((_D, _B), jnp.float32),
        scratch_types=[
            pltpu.VMEM((b_per_w,), jnp.int32),
            pltpu.VMEM((b_per_w, 2 * _D), jnp.float32),
            pltpu.VMEM((8, 128), jnp.float32),
            pltpu.SemaphoreType.DMA,
        ],
        compiler_params=pltpu.CompilerParams(needs_layout_passes=False),
    )
    def gather_kernel(table_hbm, idx_hbm, out_hbm, idx_v, rows_v, ctile, sem):
        wid = lax.axis_index("s") * num_cores + lax.axis_index("c")
        base = wid * b_per_w
        pltpu.sync_copy(idx_hbm.at[pl.ds(base, b_per_w)], idx_v)
        pltpu.async_copy(table_hbm.at[idx_v], rows_v, sem).wait()
        lane16 = lax.iota(jnp.int32, 16)
        zeros16 = lax.mul(lane16, 0)

        def do_jb(jb, carry):
            def do_tr(tr, carry2):
                for s in range(8):
                    col_idx = zeros16 + tr * 8 + s
                    for q in range(8):
                        row_idx = jb * 128 + q * 16 + lane16
                        v = plsc.load_gather(rows_v, [row_idx, col_idx])
                        ctile[s, pl.ds(q * 16, 16)] = v
                pltpu.sync_copy(
                    ctile,
                    out_hbm.at[pl.ds(tr * 8, 8), pl.ds(base + jb * 128, 128)],
                )
                return carry2
            lax.fori_loop(0, 8, do_tr, 0)
            return carry
        lax.fori_loop(0, b_per_w // 128, do_jb, 0)

    return gather_kernel


def kernel(W, indices):
    idx = indices[:, 0].astype(jnp.int32)
    Wp = jnp.pad(W, ((0, 0), (0, _D)))
    info = plsc.get_sparse_core_info()
    gather = _build_gather(info.num_cores, info.num_subcores)
    out_t = gather(Wp, idx)
    return out_t.T


# speedup vs baseline: 1.2159x; 1.2159x over previous
import functools

import jax
import jax.numpy as jnp
from jax import lax
from jax.experimental import pallas as pl
from jax.experimental.pallas import tpu as pltpu
from jax.experimental.pallas import tpu_sc as plsc

_B = 16384
_D = 64
_G = 8


@functools.cache
def _build_gather(num_cores: int, num_subcores: int):
    nw = num_cores * num_subcores
    b_per_w = _B // nw
    mesh = plsc.VectorSubcoreMesh(core_axis_name="c", subcore_axis_name="s")

    @functools.partial(
        pl.kernel,
        mesh=mesh,
        out_type=jax.ShapeDtypeStruct((_B, 2 * _D), jnp.float32),
        scratch_types=[
            pltpu.VMEM((b_per_w,), jnp.int32),
            pltpu.VMEM((b_per_w, 2 * _D), jnp.float32),
            pltpu.SemaphoreType.DMA,
        ],
    )
    def gather_kernel(table_hbm, idxt_hbm, out_hbm, idx_v, rows_v, sem):
        wid = lax.axis_index("s") * num_cores + lax.axis_index("c")
        base = wid * b_per_w
        pltpu.sync_copy(idxt_hbm.at[0, pl.ds(base, b_per_w)], idx_v)
        pltpu.async_copy(table_hbm.at[idx_v], rows_v, sem).wait()
        pltpu.sync_copy(rows_v, out_hbm.at[pl.ds(base, b_per_w)])

    return gather_kernel


def kernel(W, indices):
    idxt = indices.astype(jnp.int32).T
    Wp = jnp.pad(W, ((0, 0), (0, _D)))
    info = plsc.get_sparse_core_info()
    gather = _build_gather(info.num_cores, info.num_subcores)
    out_p = gather(Wp, idxt)
    return out_p[:, :_D]
